# Initial kernel scaffold; baseline (speedup 1.0000x reference)
import jax
import jax.numpy as jnp
from jax.experimental import pallas as pl


def kernel(x, tables):
    raise NotImplementedError("write your pallas kernel here")


# speedup vs baseline: 6.9328x; 6.9328x over previous
import functools

import jax
import jax.numpy as jnp
from jax import lax
from jax.experimental import pallas as pl
from jax.experimental.pallas import tpu as pltpu
from jax.experimental.pallas import tpu_sc as plsc

LANES = 16
NUM_CORES = 2
NUM_SUBCORES = 16
NUM_WORKERS = NUM_CORES * NUM_SUBCORES

CHUNK = 1024
GATHER = 128


def _make_gather(n_rows, vocab_total, groups, vocab, dim):
  assert n_rows % (NUM_WORKERS * CHUNK) == 0
  assert CHUNK % GATHER == 0 and LANES % groups == 0
  rows_per_worker = n_rows // NUM_WORKERS
  n_chunks = rows_per_worker // CHUNK
  mesh = plsc.VectorSubcoreMesh(core_axis_name="c", subcore_axis_name="s")

  @functools.partial(
      pl.kernel,
      mesh=mesh,
      compiler_params=pltpu.CompilerParams(use_tc_tiling_on_sc=False),
      out_type=jax.ShapeDtypeStruct((n_rows, dim), jnp.float32),
      scratch_types=[
          pltpu.VMEM((CHUNK,), jnp.int32),
          pltpu.VMEM((CHUNK, dim), jnp.float32),
          pltpu.SemaphoreType.DMA,
      ],
  )
  def k(tab_hbm, idx_hbm, out_hbm, idx_v, rows_v, sem):
    wid = lax.axis_index("s") * NUM_CORES + lax.axis_index("c")
    base0 = wid * rows_per_worker
    offs = (lax.iota(jnp.int32, LANES) % groups) * vocab

    def chunk_body(c, carry):
      base = base0 + c * CHUNK
      pltpu.sync_copy(idx_hbm.at[pl.ds(base, CHUNK)], idx_v)
      for i in range(CHUNK // LANES):
        sl = pl.ds(i * LANES, LANES)
        idx_v[sl] = idx_v[sl] + offs
      copies = []
      for j in range(CHUNK // GATHER):
        copies.append(
            pltpu.async_copy(
                tab_hbm.at[idx_v.at[pl.ds(j * GATHER, GATHER)]],
                rows_v.at[pl.ds(j * GATHER, GATHER), :],
                sem,
            ))
      for cp in copies:
        cp.wait()
      pltpu.sync_copy(rows_v, out_hbm.at[pl.ds(base, CHUNK)])
      return carry

    lax.fori_loop(0, n_chunks, chunk_body, 0)

  return k


def kernel(x, tables):
  b, s, groups = x.shape
  g2, vocab, dim = tables.shape
  n_rows = b * s * groups
  idx = x.reshape(n_rows).astype(jnp.int32)
  tab = tables.reshape(groups * vocab, dim)
  gather = _make_gather(n_rows, groups * vocab, groups, vocab, dim)
  out = gather(tab, idx)
  return out.reshape(b, s, groups * dim)


# speedup vs baseline: 7.3492x; 1.0601x over previous
import functools

import jax
import jax.numpy as jnp
from jax import lax
from jax.experimental import pallas as pl
from jax.experimental.pallas import tpu as pltpu
from jax.experimental.pallas import tpu_sc as plsc

LANES = 16
NUM_CORES = 2
NUM_SUBCORES = 16
NUM_WORKERS = NUM_CORES * NUM_SUBCORES

CHUNK = 2560
GATHER = 128


def _make_gather(n_rows, groups, vocab, dim):
  assert n_rows % (NUM_WORKERS * CHUNK) == 0
  assert CHUNK % GATHER == 0 and LANES % groups == 0
  rows_per_worker = n_rows // NUM_WORKERS
  n_chunks = rows_per_worker // CHUNK
  mesh = plsc.VectorSubcoreMesh(core_axis_name="c", subcore_axis_name="s")

  @functools.partial(
      pl.kernel,
      mesh=mesh,
      compiler_params=pltpu.CompilerParams(use_tc_tiling_on_sc=False),
      out_type=jax.ShapeDtypeStruct((n_rows, dim), jnp.float32),
      scratch_types=[
          pltpu.VMEM((rows_per_worker,), jnp.int32),
          pltpu.VMEM((2, CHUNK, dim), jnp.float32),
          pltpu.SemaphoreType.DMA,
          pltpu.SemaphoreType.DMA,
          pltpu.SemaphoreType.DMA,
          pltpu.SemaphoreType.DMA,
      ],
  )
  def k(tab_hbm, idx_hbm, out_hbm, idx_v, rows_v, gsem0, gsem1, osem0, osem1):
    wid = lax.axis_index("s") * NUM_CORES + lax.axis_index("c")
    base0 = wid * rows_per_worker
    offs = (lax.iota(jnp.int32, LANES) % groups) * vocab

    pltpu.sync_copy(idx_hbm.at[pl.ds(base0, rows_per_worker)], idx_v)

    def add_body(i, carry):
      sl = pl.ds(i * LANES, LANES)
      idx_v[sl] = idx_v[sl] + offs
      return carry

    lax.fori_loop(0, rows_per_worker // LANES, add_body, 0)

    gsems = (gsem0, gsem1)
    osems = (osem0, osem1)

    def fire_gathers(c):
      p = c & 1
      cps = []
      for j in range(CHUNK // GATHER):
        cps.append(
            pltpu.async_copy(
                tab_hbm.at[idx_v.at[pl.ds(c * CHUNK + j * GATHER, GATHER)]],
                rows_v.at[p, pl.ds(j * GATHER, GATHER), :],
                gsems[p],
            ))
      return cps

    out_cps = [None, None]

    def writeback(c, gather_cps):
      p = c & 1
      for cp in gather_cps:
        cp.wait()
      out_cps[p] = pltpu.async_copy(
          rows_v.at[p], out_hbm.at[pl.ds(base0 + c * CHUNK, CHUNK)], osems[p])

    prev = fire_gathers(0)
    for c in range(1, n_chunks):
      p = c & 1
      if out_cps[p] is not None:
        out_cps[p].wait()
      cur = fire_gathers(c)
      writeback(c - 1, prev)
      prev = cur
    writeback(n_chunks - 1, prev)
    out_cps[0].wait()
    out_cps[1].wait()

  return k


def kernel(x, tables):
  b, s, groups = x.shape
  _, vocab, dim = tables.shape
  n_rows = b * s * groups
  idx = x.reshape(n_rows).astype(jnp.int32)
  tab = tables.reshape(groups * vocab, dim)
  gather = _make_gather(n_rows, groups, vocab, dim)
  out = gather(tab, idx)
  return out.reshape(b, s, groups * dim)


# speedup vs baseline: 8.2129x; 1.1175x over previous
import functools

import jax
import jax.numpy as jnp
from jax import lax
from jax.experimental import pallas as pl
from jax.experimental.pallas import tpu as pltpu
from jax.experimental.pallas import tpu_sc as plsc

LANES = 16
NUM_WORKERS = 32
BL = 128


def _make_kernel(s_dim, bt, groups, vocab, dim):
  units = s_dim * bt
  assert units % NUM_WORKERS == 0 and dim == 16 and groups == 4
  upw = units // NUM_WORKERS
  assert upw % 2 == 0
  ct = groups * dim // 8
  mesh = plsc.VectorSubcoreMesh(core_axis_name="c", subcore_axis_name="s")

  @functools.partial(
      pl.kernel,
      mesh=mesh,
      compiler_params=pltpu.CompilerParams(use_tc_tiling_on_sc=False,
                                           needs_layout_passes=False),
      out_type=jax.ShapeDtypeStruct((s_dim, ct, bt, 8, BL), jnp.float32),
      scratch_types=[
          pltpu.VMEM((2, groups, BL), jnp.int32),
          pltpu.VMEM((2, groups, BL, dim), jnp.float32),
          pltpu.VMEM((2, ct, 8, BL), jnp.float32),
          pltpu.SemaphoreType.DMA,
          pltpu.SemaphoreType.DMA,
          pltpu.SemaphoreType.DMA,
          pltpu.SemaphoreType.DMA,
          pltpu.SemaphoreType.DMA,
          pltpu.SemaphoreType.DMA,
      ],
  )
  def k(tab_hbm, xv_hbm, out_hbm, idx_v, rows_v, trans_v,
        isem0, isem1, gsem0, gsem1, osem0, osem1):
    isems = (isem0, isem1)
    gsems = (gsem0, gsem1)
    osems = (osem0, osem1)
    wid = lax.axis_index("s") * 2 + lax.axis_index("c")
    u0 = wid * upw

    iota = lax.iota(jnp.int32, LANES)
    tok_idx = [iota + 16 * t for t in range(BL // LANES)]
    g_idx = [jnp.full((LANES,), g, jnp.int32) for g in range(groups)]
    c_idx = [jnp.full((LANES,), c, jnp.int32) for c in range(dim)]

    def fire_idx(u, b):
      return pltpu.async_copy(
          xv_hbm.at[pl.ds(u * groups, groups)], idx_v.at[b], isems[b])

    def add_offsets(b):
      for g in range(1, groups):
        for j in range(BL // LANES):
          sl = pl.ds(j * LANES, LANES)
          idx_v[b, g, sl] = idx_v[b, g, sl] + jnp.int32(g * vocab)

    def fire_gathers(b):
      for g in range(groups):
        pltpu.async_copy(tab_hbm.at[idx_v.at[b, g]], rows_v.at[b, g],
                         gsems[b])

    def drain_gathers(b):
      for g in range(groups):
        pltpu.make_async_copy(tab_hbm.at[idx_v.at[b, g]], rows_v.at[b, g],
                              gsems[b]).wait()

    def transpose_unit(b):
      src = rows_v.at[b]
      for g in range(groups):
        for c in range(dim):
          row = 2 * g + c // 8
          for t in range(BL // LANES):
            vals = plsc.load_gather(src, [g_idx[g], tok_idx[t], c_idx[c]])
            trans_v[b, row, c % 8, pl.ds(t * LANES, LANES)] = vals

    def out_slice(u):
      s = u // bt
      btile = lax.rem(u, jnp.int32(bt))
      return out_hbm.at[s, :, btile]

    def fire_out(u, b):
      pltpu.async_copy(trans_v.at[b], out_slice(u), osems[b])

    def drain_out(u, b):
      pltpu.make_async_copy(trans_v.at[b], out_slice(u), osems[b]).wait()

    fire_idx(u0, 0)

    def body(i, carry):
      for b in range(2):
        u = u0 + i + b
        pltpu.make_async_copy(
            xv_hbm.at[pl.ds(u * groups, groups)], idx_v.at[b],
            isems[b]).wait()
        add_offsets(b)

        @pl.when(i + b >= 2)
        def _():
          drain_out(u, b)

        fire_gathers(b)

        @pl.when(i + b == 0)
        def _():
          fire_idx(u + 1, 1 - b)

        @pl.when(i + b >= 1)
        def _():
          drain_gathers(1 - b)

          @pl.when(i + b + 1 < upw)
          def _():
            fire_idx(u + 1, 1 - b)

          transpose_unit(1 - b)
          fire_out(u - 1, 1 - b)

      return carry

    lax.fori_loop(0, upw // 2, lambda i, c: body(2 * i, c), 0, unroll=False)

    u_last = u0 + upw - 1
    drain_gathers(1)
    transpose_unit(1)
    fire_out(u_last, 1)
    drain_out(u_last - 1, 0)
    drain_out(u_last, 1)

  return k


def kernel(x, tables):
  b, s_dim, groups = x.shape
  _, vocab, dim = tables.shape
  bt = b // BL
  xv = (x.transpose((1, 0, 2))
        .reshape(s_dim, bt, BL, groups)
        .transpose((0, 1, 3, 2))
        .reshape(s_dim * bt * groups, BL)
        .astype(jnp.int32))
  tab = tables.reshape(groups * vocab, dim)
  k = _make_kernel(s_dim, bt, groups, vocab, dim)
  out5 = k(tab, xv)
  return (out5.transpose((2, 4, 0, 1, 3))
          .reshape(b, s_dim, groups * dim))


# speedup vs baseline: 15.5143x; 1.8890x over previous
import functools

import jax
import jax.numpy as jnp
from jax import lax
from jax.experimental import pallas as pl
from jax.experimental.pallas import tpu as pltpu
from jax.experimental.pallas import tpu_sc as plsc

LANES = 16
NUM_WORKERS = 32
BL = 128


def _make_kernel(s_dim, bt, groups, vocab, dim):
  assert groups * dim == 2 * NUM_WORKERS and dim == 16
  ct = groups * dim // 8
  mesh = plsc.VectorSubcoreMesh(core_axis_name="c", subcore_axis_name="s")

  @functools.partial(
      pl.kernel,
      mesh=mesh,
      compiler_params=pltpu.CompilerParams(use_tc_tiling_on_sc=False,
                                           needs_layout_passes=False),
      out_type=jax.ShapeDtypeStruct((s_dim, ct, bt, 8 * BL), jnp.float32),
      scratch_types=[
          pltpu.VMEM((vocab,), jnp.float32),
          pltpu.VMEM((2, bt, BL), jnp.int32),
          pltpu.VMEM((2, bt, BL), jnp.float32),
          pltpu.SemaphoreType.DMA,
          pltpu.SemaphoreType.DMA,
          pltpu.SemaphoreType.DMA,
          pltpu.SemaphoreType.DMA,
      ],
  )
  def k(tab_hbm, xv_hbm, out_hbm, drow_v, idx_v, obuf_v,
        isem0, isem1, osem0, osem1):
    isems = (isem0, isem1)
    osems = (osem0, osem1)
    wid = lax.axis_index("s") * 2 + lax.axis_index("c")
    g = wid // 8

    def fire_idx(s, b):
      return pltpu.async_copy(
          xv_hbm.at[pl.ds(s * bt, bt), g, :], idx_v.at[b], isems[b])

    def wait_idx(s, b):
      pltpu.make_async_copy(
          xv_hbm.at[pl.ds(s * bt, bt), g, :], idx_v.at[b], isems[b]).wait()

    def gather_s(b):
      for btile in range(bt):
        for t in range(BL // LANES):
          sl = pl.ds(t * LANES, LANES)
          vals = plsc.load_gather(drow_v, [idx_v[b, btile, sl]])
          obuf_v[b, btile, sl] = vals

    def out_ref(s, row, c8):
      return out_hbm.at[s, row, :, pl.ds(c8 * BL, BL)]

    def fire_out(s, row, c8, b):
      pltpu.async_copy(obuf_v.at[b], out_ref(s, row, c8), osems[b])

    def drain_out(s, row, c8, b):
      pltpu.make_async_copy(obuf_v.at[b], out_ref(s, row, c8),
                            osems[b]).wait()

    for dd in range(2):
      d_local = lax.rem(2 * wid + dd, jnp.int32(dim))
      row = 2 * g + d_local // 8
      c8 = lax.rem(d_local, jnp.int32(8))

      pltpu.sync_copy(tab_hbm.at[g, d_local], drow_v)
      fire_idx(0, 0)

      def body(i, carry):
        for b in range(2):
          s = i + b
          wait_idx(s, b)

          @pl.when(s + 1 < s_dim)
          def _():
            fire_idx(s + 1, 1 - b)

          @pl.when(s >= 2)
          def _():
            drain_out(s, row, c8, b)

          gather_s(b)
          fire_out(s, row, c8, b)
        return carry

      lax.fori_loop(0, s_dim // 2, lambda i, c: body(2 * i, c), 0,
                    unroll=False)
      drain_out(0, row, c8, 0)
      drain_out(0, row, c8, 1)

  return k


def kernel(x, tables):
  b, s_dim, groups = x.shape
  _, vocab, dim = tables.shape
  bt = b // BL
  xv = (x.transpose((1, 0, 2))
        .reshape(s_dim, bt, BL, groups)
        .transpose((0, 1, 3, 2))
        .reshape(s_dim * bt, groups, BL)
        .astype(jnp.int32))
  tabt = tables.transpose((0, 2, 1))
  k = _make_kernel(s_dim, bt, groups, vocab, dim)
  out5 = k(tabt, xv)
  return (out5.reshape(s_dim, 2 * groups, bt, 8, BL)
          .transpose((2, 4, 0, 1, 3))
          .reshape(b, s_dim, groups * dim))


# speedup vs baseline: 20.3269x; 1.3102x over previous
import functools

import jax
import jax.numpy as jnp
from jax import lax
from jax.experimental import pallas as pl
from jax.experimental.pallas import tpu as pltpu
from jax.experimental.pallas import tpu_sc as plsc

LANES = 16
NUM_WORKERS = 32
BL = 128


def _make_kernel(s_dim, bt, groups, vocab, dim):
  assert groups * dim == 2 * NUM_WORKERS and dim == 16
  ct = groups * dim // 8
  mesh = plsc.VectorSubcoreMesh(core_axis_name="c", subcore_axis_name="s")

  @functools.partial(
      pl.kernel,
      mesh=mesh,
      compiler_params=pltpu.CompilerParams(use_tc_tiling_on_sc=False,
                                           needs_layout_passes=False),
      out_type=jax.ShapeDtypeStruct((s_dim, ct, bt, 8 * BL), jnp.float32),
      scratch_types=[
          pltpu.VMEM((vocab,), jnp.float32),
          pltpu.VMEM((2, bt, BL), jnp.int32),
          pltpu.VMEM((2, bt, BL), jnp.float32),
          pltpu.SemaphoreType.DMA,
          pltpu.SemaphoreType.DMA,
          pltpu.SemaphoreType.DMA,
          pltpu.SemaphoreType.DMA,
      ],
  )
  def k(tab_hbm, xv_hbm, out_hbm, drow_v, idx_v, obuf_v,
        isem0, isem1, osem0, osem1):
    isems = (isem0, isem1)
    osems = (osem0, osem1)
    wid = lax.axis_index("s") * 2 + lax.axis_index("c")
    g = wid // 8

    def fire_idx(s, b):
      return pltpu.async_copy(
          xv_hbm.at[pl.ds(s * bt, bt), g, :], idx_v.at[b], isems[b])

    def wait_idx(s, b):
      pltpu.make_async_copy(
          xv_hbm.at[pl.ds(s * bt, bt), g, :], idx_v.at[b], isems[b]).wait()

    def gather_s(b):
      nt = BL // LANES
      for btile in range(bt):
        idxs = [idx_v[b, btile, pl.ds(t * LANES, LANES)] for t in range(nt)]
        vals = [plsc.load_gather(drow_v, [iv]) for iv in idxs]
        for t in range(nt):
          obuf_v[b, btile, pl.ds(t * LANES, LANES)] = vals[t]

    def out_ref(s, row, c8):
      return out_hbm.at[s, row, :, pl.ds(c8 * BL, BL)]

    def fire_out(s, row, c8, b):
      pltpu.async_copy(obuf_v.at[b], out_ref(s, row, c8), osems[b])

    def drain_out(s, row, c8, b):
      pltpu.make_async_copy(obuf_v.at[b], out_ref(s, row, c8),
                            osems[b]).wait()

    for dd in range(2):
      d_local = lax.rem(2 * wid + dd, jnp.int32(dim))
      row = 2 * g + d_local // 8
      c8 = lax.rem(d_local, jnp.int32(8))

      pltpu.sync_copy(tab_hbm.at[g, d_local], drow_v)
      fire_idx(0, 0)

      def body(i, carry):
        for b in range(2):
          s = i + b
          wait_idx(s, b)

          @pl.when(s + 1 < s_dim)
          def _():
            fire_idx(s + 1, 1 - b)

          @pl.when(s >= 2)
          def _():
            drain_out(s, row, c8, b)

          gather_s(b)
          fire_out(s, row, c8, b)
        return carry

      lax.fori_loop(0, s_dim // 2, lambda i, c: body(2 * i, c), 0,
                    unroll=False)
      drain_out(0, row, c8, 0)
      drain_out(0, row, c8, 1)

  return k


def kernel(x, tables):
  b, s_dim, groups = x.shape
  _, vocab, dim = tables.shape
  bt = b // BL
  xv = (x.transpose((1, 0, 2))
        .reshape(s_dim, bt, BL, groups)
        .transpose((0, 1, 3, 2))
        .reshape(s_dim * bt, groups, BL)
        .astype(jnp.int32))
  tabt = tables.transpose((0, 2, 1))
  k = _make_kernel(s_dim, bt, groups, vocab, dim)
  out5 = k(tabt, xv)
  return (out5.reshape(s_dim, 2 * groups, bt, 8, BL)
          .transpose((2, 4, 0, 1, 3))
          .reshape(b, s_dim, groups * dim))
